# Initial kernel scaffold; baseline (speedup 1.0000x reference)
import jax
import jax.numpy as jnp
from jax.experimental import pallas as pl


def kernel(input_ids, W, P, gamma, beta):
    raise NotImplementedError("write your pallas kernel here")


# speedup vs baseline: 3.9501x; 3.9501x over previous
import functools

import jax
import jax.numpy as jnp
from jax import lax
from jax.experimental import pallas as pl
from jax.experimental.pallas import tpu as pltpu
from jax.experimental.pallas import tpu_sc as plsc

B = 1024
L = 200
DIM = 128
EPS = 1e-12

NC = 2
NS = 16
NW = NC * NS

TOTAL_ROWS = B * L
ROWS_PER_W = TOTAL_ROWS // NW
CHUNK = 128
CHUNKS_PER_W = ROWS_PER_W // CHUNK


def _sc_gather(ids3d, W):
    mesh = plsc.VectorSubcoreMesh(core_axis_name="c", subcore_axis_name="s")

    @functools.partial(
        pl.kernel,
        mesh=mesh,
        out_type=jax.ShapeDtypeStruct((TOTAL_ROWS, DIM), jnp.float32),
        scratch_types=[
            pltpu.VMEM((CHUNKS_PER_W, CHUNK), jnp.int32),
            pltpu.VMEM((CHUNK, DIM), jnp.float32),
            pltpu.SemaphoreType.DMA,
        ],
    )
    def k(ids_hbm, w_hbm, out_hbm, idx_v, rows_v, sem):
        wid = lax.axis_index("s") * NC + lax.axis_index("c")
        pltpu.sync_copy(ids_hbm.at[wid], idx_v)
        row_base = wid * ROWS_PER_W

        def body(j, carry):
            pltpu.async_copy(w_hbm.at[idx_v.at[j]], rows_v, sem).wait()
            off = pl.multiple_of(row_base + j * CHUNK, CHUNK)
            pltpu.sync_copy(rows_v, out_hbm.at[pl.ds(off, CHUNK)])
            return carry

        lax.fori_loop(0, CHUNKS_PER_W, body, 0)

    return k(ids3d, W)


def _ln_body(emb_ref, p_ref, g_ref, b_ref, out_ref):
    x = emb_ref[...] + p_ref[...][None, :, :]
    mean = jnp.mean(x, axis=-1, keepdims=True)
    var = jnp.mean((x - mean) ** 2, axis=-1, keepdims=True)
    normed = (x - mean) / jnp.sqrt(var + EPS)
    out_ref[...] = normed * g_ref[...][None, :, :] + b_ref[...][None, :, :]


def _tc_layernorm(emb3, P200, gamma2, beta2):
    bs = 16
    grid = (B // bs,)
    return pl.pallas_call(
        _ln_body,
        grid=grid,
        in_specs=[
            pl.BlockSpec((bs, L, DIM), lambda i: (i, 0, 0)),
            pl.BlockSpec((L, DIM), lambda i: (0, 0)),
            pl.BlockSpec((1, DIM), lambda i: (0, 0)),
            pl.BlockSpec((1, DIM), lambda i: (0, 0)),
        ],
        out_specs=pl.BlockSpec((bs, L, DIM), lambda i: (i, 0, 0)),
        out_shape=jax.ShapeDtypeStruct((B, L, DIM), jnp.float32),
    )(emb3, P200, gamma2, beta2)


def kernel(input_ids, W, P, gamma, beta):
    ids3d = input_ids.reshape(NW, CHUNKS_PER_W, CHUNK)
    emb_flat = _sc_gather(ids3d, W)
    emb3 = emb_flat.reshape(B, L, DIM)
    return _tc_layernorm(emb3, P[:L], gamma.reshape(1, DIM), beta.reshape(1, DIM))
